# baseline (speedup 1.0000x reference)
import numpy as np
import jax
import jax.numpy as jnp
from jax import lax
from jax.experimental import pallas as pl


def _np_threefry2x32(k0, k1, x0, x1):
    def rotl(x, d):
        return ((x << np.uint32(d)) | (x >> np.uint32(32 - d))).astype(np.uint32)
    ks = [np.uint32(k0), np.uint32(k1),
          np.uint32(np.uint32(k0) ^ np.uint32(k1) ^ np.uint32(0x1BD11BDA))]
    x0 = (x0 + ks[0]).astype(np.uint32)
    x1 = (x1 + ks[1]).astype(np.uint32)
    rots = [[13, 15, 26, 6], [17, 29, 16, 24]]
    for i in range(5):
        for r in rots[i % 2]:
            x0 = (x0 + x1).astype(np.uint32)
            x1 = rotl(x1, r)
            x1 = (x0 ^ x1).astype(np.uint32)
        x0 = (x0 + ks[(i + 1) % 3]).astype(np.uint32)
        x1 = (x1 + ks[(i + 2) % 3] + np.uint32(i + 1)).astype(np.uint32)
    return x0, x1


def _child_key(seed_hi, seed_lo, i):
    a, b = _np_threefry2x32(seed_hi, seed_lo,
                            np.array([0], np.uint32), np.array([i], np.uint32))
    return int(a[0]), int(b[0])


_K1 = _child_key(0, 42, 0)
_K2 = _child_key(0, 42, 1)

_TINY = float(np.finfo(np.float32).tiny)
_U2_MIN = np.float32(1e-5)
_U2_SPAN = np.float32(np.float32(1.0 - 1e-5) - np.float32(1e-5))

_ROT = ((13, 15, 26, 6), (17, 29, 16, 24))


def _wrap_i32(v):
    v &= 0xFFFFFFFF
    return v - (1 << 32) if v >= (1 << 31) else v


def _rotl(x, r):
    return lax.shift_left(x, jnp.int32(r)) | lax.shift_right_logical(x, jnp.int32(32 - r))


def _threefry_bits(kpair, x1):
    k0, k1 = kpair
    ks = (_wrap_i32(k0), _wrap_i32(k1), _wrap_i32(k0 ^ k1 ^ 0x1BD11BDA))
    x0 = jnp.full_like(x1, jnp.int32(ks[0]))
    x1 = x1 + jnp.int32(ks[1])
    for i in range(5):
        for r in _ROT[i % 2]:
            x0 = x0 + x1
            x1 = _rotl(x1, r)
            x1 = x0 ^ x1
        x0 = x0 + jnp.int32(ks[(i + 1) % 3])
        x1 = x1 + jnp.int32(_wrap_i32(ks[(i + 2) % 3] + i + 1))
    return x0 ^ x1


def _bits_to_unit(bits):
    fb = lax.shift_right_logical(bits, jnp.int32(9)) | jnp.int32(0x3F800000)
    return lax.bitcast_convert_type(fb, jnp.float32) - jnp.float32(1.0)


def _body(l_ref, out_ref, *, n, T):
    b = pl.program_id(0)
    lb = l_ref[0]
    logits = lb[0:n, :]

    g_iota = lax.broadcasted_iota(jnp.int32, (n, T), 0)
    t_iota = lax.broadcasted_iota(jnp.int32, (n, T), 1)
    flat = (b * n + g_iota) * T + t_iota

    bits = _threefry_bits(_K1, flat)
    uf = _bits_to_unit(bits)
    ug = jnp.maximum(jnp.float32(_TINY),
                     uf * jnp.float32(1.0 - _TINY) + jnp.float32(_TINY))
    gum = -jnp.log(-jnp.log(ug))
    val = logits + gum

    m = jnp.max(val, axis=0, keepdims=True)
    sel = jnp.min(jnp.where(val == m, g_iota, jnp.int32(n)),
                  axis=0, keepdims=True)
    onehot = (g_iota == sel).astype(jnp.float32)
    mean_raw = jnp.sum(lb[n:2 * n, :] * onehot, axis=0, keepdims=True)
    lv_raw = jnp.sum(lb[2 * n:3 * n, :] * onehot, axis=0, keepdims=True)

    sel_mean = jnp.tanh(mean_raw)
    sel_lv = jnp.float32(-7.0) * jax.nn.sigmoid(lv_raw)

    t1 = lax.broadcasted_iota(jnp.int32, (1, T), 1)
    bits2 = _threefry_bits(_K2, b * T + t1)
    uf2 = _bits_to_unit(bits2)
    u2 = jnp.maximum(_U2_MIN, uf2 * _U2_SPAN + _U2_MIN)

    noise = jnp.exp(sel_lv) * (jnp.log(u2) - jnp.log(jnp.float32(1.0) - u2))
    x = jnp.clip(sel_mean + noise, -1.0, 1.0)
    y = jnp.round((x + jnp.float32(1.0)) * jnp.float32(127.5))
    out_ref[0] = y.astype(jnp.int32)


def kernel(l):
    B, C, T = l.shape
    n = C // 3
    import functools
    out = pl.pallas_call(
        functools.partial(_body, n=n, T=T),
        grid=(B,),
        in_specs=[pl.BlockSpec((1, C, T), lambda b: (b, 0, 0))],
        out_specs=pl.BlockSpec((1, 1, T), lambda b: (b, 0, 0)),
        out_shape=jax.ShapeDtypeStruct((B, 1, T), jnp.int32),
    )(l)
    return out.reshape(B, T)


# speedup vs baseline: 1.0468x; 1.0468x over previous
import numpy as np
import jax
import jax.numpy as jnp
from jax import lax
from jax.experimental import pallas as pl


def _np_threefry2x32(k0, k1, x0, x1):
    def rotl(x, d):
        return ((x << np.uint32(d)) | (x >> np.uint32(32 - d))).astype(np.uint32)
    ks = [np.uint32(k0), np.uint32(k1),
          np.uint32(np.uint32(k0) ^ np.uint32(k1) ^ np.uint32(0x1BD11BDA))]
    x0 = (x0 + ks[0]).astype(np.uint32)
    x1 = (x1 + ks[1]).astype(np.uint32)
    rots = [[13, 15, 26, 6], [17, 29, 16, 24]]
    for i in range(5):
        for r in rots[i % 2]:
            x0 = (x0 + x1).astype(np.uint32)
            x1 = rotl(x1, r)
            x1 = (x0 ^ x1).astype(np.uint32)
        x0 = (x0 + ks[(i + 1) % 3]).astype(np.uint32)
        x1 = (x1 + ks[(i + 2) % 3] + np.uint32(i + 1)).astype(np.uint32)
    return x0, x1


def _child_key(seed_hi, seed_lo, i):
    a, b = _np_threefry2x32(seed_hi, seed_lo,
                            np.array([0], np.uint32), np.array([i], np.uint32))
    return int(a[0]), int(b[0])


_K1 = _child_key(0, 42, 0)
_K2 = _child_key(0, 42, 1)

_TINY = float(np.finfo(np.float32).tiny)
_U2_MIN = np.float32(1e-5)
_U2_SPAN = np.float32(np.float32(1.0 - 1e-5) - np.float32(1e-5))

_ROT = ((13, 15, 26, 6), (17, 29, 16, 24))


def _wrap_i32(v):
    v &= 0xFFFFFFFF
    return v - (1 << 32) if v >= (1 << 31) else v


def _rotl(x, r):
    return lax.shift_left(x, jnp.int32(r)) | lax.shift_right_logical(x, jnp.int32(32 - r))


def _threefry_bits(kpair, x1):
    k0, k1 = kpair
    ks = (_wrap_i32(k0), _wrap_i32(k1), _wrap_i32(k0 ^ k1 ^ 0x1BD11BDA))
    x0 = jnp.full_like(x1, jnp.int32(ks[0]))
    x1 = x1 + jnp.int32(ks[1])
    for i in range(5):
        for r in _ROT[i % 2]:
            x0 = x0 + x1
            x1 = _rotl(x1, r)
            x1 = x0 ^ x1
        x0 = x0 + jnp.int32(ks[(i + 1) % 3])
        x1 = x1 + jnp.int32(_wrap_i32(ks[(i + 2) % 3] + i + 1))
    return x0 ^ x1


def _bits_to_unit(bits):
    fb = lax.shift_right_logical(bits, jnp.int32(9)) | jnp.int32(0x3F800000)
    return lax.bitcast_convert_type(fb, jnp.float32) - jnp.float32(1.0)


def _body(l_ref, out_ref, *, n, S, W):
    T = S * W
    b = pl.program_id(0)
    lb = l_ref[0]
    logits = lb[0:n]

    g_iota = lax.broadcasted_iota(jnp.int32, (n, S, W), 0)
    s_iota = lax.broadcasted_iota(jnp.int32, (n, S, W), 1)
    c_iota = lax.broadcasted_iota(jnp.int32, (n, S, W), 2)
    flat = (b * n + g_iota) * T + s_iota * W + c_iota

    bits = _threefry_bits(_K1, flat)
    ug = jnp.maximum(_bits_to_unit(bits), jnp.float32(_TINY))
    gum = -jnp.log(-jnp.log(ug))
    val = logits + gum

    m = jnp.max(val, axis=0, keepdims=True)
    sel = jnp.min(jnp.where(val == m, g_iota, jnp.int32(n)),
                  axis=0, keepdims=True)
    onehot = (g_iota == sel).astype(jnp.float32)
    mean_raw = jnp.sum(lb[n:2 * n] * onehot, axis=0)
    lv_raw = jnp.sum(lb[2 * n:3 * n] * onehot, axis=0)

    sel_mean = jnp.tanh(mean_raw)
    sel_lv = jnp.float32(-7.0) * jax.nn.sigmoid(lv_raw)

    s2 = lax.broadcasted_iota(jnp.int32, (S, W), 0)
    c2 = lax.broadcasted_iota(jnp.int32, (S, W), 1)
    bits2 = _threefry_bits(_K2, b * T + s2 * W + c2)
    uf2 = _bits_to_unit(bits2)
    u2 = jnp.maximum(_U2_MIN, uf2 * _U2_SPAN + _U2_MIN)

    noise = jnp.exp(sel_lv) * (jnp.log(u2) - jnp.log(jnp.float32(1.0) - u2))
    x = jnp.clip(sel_mean + noise, -1.0, 1.0)
    y = jnp.round((x + jnp.float32(1.0)) * jnp.float32(127.5))
    out_ref[0] = y.astype(jnp.int32)


def kernel(l):
    B, C, T = l.shape
    n = C // 3
    S, W = 8, T // 8
    import functools
    out = pl.pallas_call(
        functools.partial(_body, n=n, S=S, W=W),
        grid=(B,),
        in_specs=[pl.BlockSpec((1, C, S, W), lambda b: (b, 0, 0, 0))],
        out_specs=pl.BlockSpec((1, S, W), lambda b: (b, 0, 0)),
        out_shape=jax.ShapeDtypeStruct((B, S, W), jnp.int32),
    )(l.reshape(B, C, S, W))
    return out.reshape(B, T)


# speedup vs baseline: 1.0812x; 1.0328x over previous
import functools
import numpy as np
import jax
import jax.numpy as jnp
from jax import lax
from jax.experimental import pallas as pl


def _np_threefry2x32(k0, k1, x0, x1):
    def rotl(x, d):
        return ((x << np.uint32(d)) | (x >> np.uint32(32 - d))).astype(np.uint32)
    ks = [np.uint32(k0), np.uint32(k1),
          np.uint32(np.uint32(k0) ^ np.uint32(k1) ^ np.uint32(0x1BD11BDA))]
    x0 = (x0 + ks[0]).astype(np.uint32)
    x1 = (x1 + ks[1]).astype(np.uint32)
    rots = [[13, 15, 26, 6], [17, 29, 16, 24]]
    for i in range(5):
        for r in rots[i % 2]:
            x0 = (x0 + x1).astype(np.uint32)
            x1 = rotl(x1, r)
            x1 = (x0 ^ x1).astype(np.uint32)
        x0 = (x0 + ks[(i + 1) % 3]).astype(np.uint32)
        x1 = (x1 + ks[(i + 2) % 3] + np.uint32(i + 1)).astype(np.uint32)
    return x0, x1


def _child_key(seed_hi, seed_lo, i):
    a, b = _np_threefry2x32(seed_hi, seed_lo,
                            np.array([0], np.uint32), np.array([i], np.uint32))
    return int(a[0]), int(b[0])


_K1 = _child_key(0, 42, 0)
_K2 = _child_key(0, 42, 1)

_TINY = float(np.finfo(np.float32).tiny)
_U2_MIN = np.float32(1e-5)
_U2_SPAN = np.float32(np.float32(1.0 - 1e-5) - np.float32(1e-5))

_ROT = ((13, 15, 26, 6), (17, 29, 16, 24))


def _wrap_i32(v):
    v &= 0xFFFFFFFF
    return v - (1 << 32) if v >= (1 << 31) else v


def _rotl(x, r):
    return lax.shift_left(x, jnp.int32(r)) | lax.shift_right_logical(x, jnp.int32(32 - r))


def _key_consts(kpair):
    k0, k1 = kpair
    return (_wrap_i32(k0), _wrap_i32(k1), _wrap_i32(k0 ^ k1 ^ 0x1BD11BDA))


_KS1 = _key_consts(_K1)
_KS2 = _key_consts(_K2)


def _threefry_bits(ks, x1):
    ksl = tuple(jnp.int32(k) if isinstance(k, (int, np.integer)) else k
                for k in ks)
    x0 = jnp.zeros_like(x1) + ksl[0]
    x1 = x1 + ksl[1]
    for i in range(5):
        for r in _ROT[i % 2]:
            x0 = x0 + x1
            x1 = _rotl(x1, r)
            x1 = x0 ^ x1
        x0 = x0 + ksl[(i + 1) % 3]
        x1 = x1 + ksl[(i + 2) % 3] + jnp.int32(i + 1)
    return x0 ^ x1


def _bits_to_unit(bits):
    fb = lax.shift_right_logical(bits, jnp.int32(9)) | jnp.int32(0x3F800000)
    return lax.bitcast_convert_type(fb, jnp.float32) - jnp.float32(1.0)


def _body(l_ref, out_ref, *, n, T, TW):
    b = pl.program_id(0)
    s = pl.program_id(1)
    lb = l_ref[0]
    logits = lb[0:n]

    rows = n + 1
    g_iota = lax.broadcasted_iota(jnp.int32, (rows, TW), 0)
    c_iota = lax.broadcasted_iota(jnp.int32, (rows, TW), 1)
    t_abs = s * TW + c_iota
    is_g = g_iota < n
    flat = jnp.where(is_g, (b * n + g_iota) * T, jnp.int32(b * T)) + t_abs
    ks = tuple(jnp.where(is_g, jnp.int32(a), jnp.int32(c))
               for a, c in zip(_KS1, _KS2))
    bits = _threefry_bits(ks, flat)
    uf = _bits_to_unit(bits)

    ug = jnp.maximum(uf[0:n], jnp.float32(_TINY))
    gum = -jnp.log(-jnp.log(ug))
    val = logits + gum

    gi = g_iota[0:n]
    m = jnp.max(val, axis=0, keepdims=True)
    sel = jnp.min(jnp.where(val == m, gi, jnp.int32(n)),
                  axis=0, keepdims=True)
    onehot = (gi == sel).astype(jnp.float32)
    mean_raw = jnp.sum(lb[n:2 * n] * onehot, axis=0, keepdims=True)
    lv_raw = jnp.sum(lb[2 * n:3 * n] * onehot, axis=0, keepdims=True)

    sel_mean = jnp.tanh(mean_raw)
    sel_lv = jnp.float32(-7.0) * jax.nn.sigmoid(lv_raw)

    u2 = jnp.maximum(_U2_MIN, uf[n:n + 1] * _U2_SPAN + _U2_MIN)
    noise = jnp.exp(sel_lv) * (jnp.log(u2) - jnp.log(jnp.float32(1.0) - u2))
    x = jnp.clip(sel_mean + noise, -1.0, 1.0)
    y = jnp.round((x + jnp.float32(1.0)) * jnp.float32(127.5))
    out_ref[0] = y.astype(jnp.int32)


def kernel(l):
    B, C, T = l.shape
    n = C // 3
    TW = 2048
    out = pl.pallas_call(
        functools.partial(_body, n=n, T=T, TW=TW),
        grid=(B, T // TW),
        in_specs=[pl.BlockSpec((1, C, TW), lambda b, s: (b, 0, s))],
        out_specs=pl.BlockSpec((1, 1, TW), lambda b, s: (b, 0, s)),
        out_shape=jax.ShapeDtypeStruct((B, 1, T), jnp.int32),
    )(l)
    return out.reshape(B, T)


# speedup vs baseline: 2.0068x; 1.8561x over previous
import functools
import numpy as np
import jax
import jax.numpy as jnp
from jax import lax
from jax.experimental import pallas as pl


def _np_threefry2x32(k0, k1, x0, x1):
    def rotl(x, d):
        return ((x << np.uint32(d)) | (x >> np.uint32(32 - d))).astype(np.uint32)
    ks = [np.uint32(k0), np.uint32(k1),
          np.uint32(np.uint32(k0) ^ np.uint32(k1) ^ np.uint32(0x1BD11BDA))]
    x0 = (x0 + ks[0]).astype(np.uint32)
    x1 = (x1 + ks[1]).astype(np.uint32)
    rots = [[13, 15, 26, 6], [17, 29, 16, 24]]
    for i in range(5):
        for r in rots[i % 2]:
            x0 = (x0 + x1).astype(np.uint32)
            x1 = rotl(x1, r)
            x1 = (x0 ^ x1).astype(np.uint32)
        x0 = (x0 + ks[(i + 1) % 3]).astype(np.uint32)
        x1 = (x1 + ks[(i + 2) % 3] + np.uint32(i + 1)).astype(np.uint32)
    return x0, x1


def _child_key(seed_hi, seed_lo, i):
    a, b = _np_threefry2x32(seed_hi, seed_lo,
                            np.array([0], np.uint32), np.array([i], np.uint32))
    return int(a[0]), int(b[0])


_K1 = _child_key(0, 42, 0)
_K2 = _child_key(0, 42, 1)

_TINY = float(np.finfo(np.float32).tiny)
_U2_MIN = np.float32(1e-5)
_U2_SPAN = np.float32(np.float32(1.0 - 1e-5) - np.float32(1e-5))

_ROT = ((13, 15, 26, 6), (17, 29, 16, 24))


def _wrap_i32(v):
    v &= 0xFFFFFFFF
    return v - (1 << 32) if v >= (1 << 31) else v


def _rotl(x, r):
    return lax.shift_left(x, jnp.int32(r)) | lax.shift_right_logical(x, jnp.int32(32 - r))


def _key_sched(kpair):
    k0, k1 = kpair
    ks = (_wrap_i32(k0), _wrap_i32(k1), _wrap_i32(k0 ^ k1 ^ 0x1BD11BDA))
    inj = [(ks[(i + 1) % 3], _wrap_i32(ks[(i + 2) % 3] + i + 1)) for i in range(5)]
    return ks[0], ks[1], inj


_SCHED1 = _key_sched(_K1)
_SCHED2 = _key_sched(_K2)


def _threefry_bits(sched, x1):
    ks0, ks1, inj = sched
    x0 = jnp.full_like(x1, jnp.int32(ks0))
    x1 = x1 + jnp.int32(ks1)
    for i in range(5):
        for r in _ROT[i % 2]:
            x0 = x0 + x1
            x1 = _rotl(x1, r)
            x1 = x0 ^ x1
        x0 = x0 + jnp.int32(inj[i][0])
        x1 = x1 + jnp.int32(inj[i][1])
    return x0 ^ x1


def _bits_to_unit(bits):
    fb = lax.shift_right_logical(bits, jnp.int32(9)) | jnp.int32(0x3F800000)
    return lax.bitcast_convert_type(fb, jnp.float32) - jnp.float32(1.0)


def _body(l_ref, out_ref, *, n, T, BW, TW):
    i = pl.program_id(0)
    j = pl.program_id(1)
    lb = l_ref[...]
    logits = lb[0:n]

    g_iota = lax.broadcasted_iota(jnp.int32, (n, BW, TW), 0)
    b_iota = lax.broadcasted_iota(jnp.int32, (n, BW, TW), 1)
    c_iota = lax.broadcasted_iota(jnp.int32, (n, BW, TW), 2)
    t_abs = j * TW + c_iota
    flat = ((i * BW + b_iota) * n + g_iota) * T + t_abs

    bits = _threefry_bits(_SCHED1, flat)
    ug = jnp.maximum(_bits_to_unit(bits), jnp.float32(_TINY))
    gum = -jnp.log(-jnp.log(ug))
    val = logits + gum

    m = jnp.max(val, axis=0)
    sel = jnp.min(jnp.where(val == m[None], g_iota, jnp.int32(n)),
                  axis=0)
    onehot = g_iota == sel[None]
    zero = jnp.float32(0.0)
    mean_raw = jnp.sum(jnp.where(onehot, lb[n:2 * n], zero), axis=0)
    lv_raw = jnp.sum(jnp.where(onehot, lb[2 * n:3 * n], zero), axis=0)

    sel_mean = jnp.tanh(mean_raw)
    sel_lv = jnp.float32(-7.0) * jax.nn.sigmoid(lv_raw)

    b2 = lax.broadcasted_iota(jnp.int32, (BW, TW), 0)
    c2 = lax.broadcasted_iota(jnp.int32, (BW, TW), 1)
    bits2 = _threefry_bits(_SCHED2, (i * BW + b2) * T + j * TW + c2)
    u2 = jnp.maximum(_U2_MIN, _bits_to_unit(bits2) * _U2_SPAN + _U2_MIN)

    noise = jnp.exp(sel_lv) * (jnp.log(u2) - jnp.log(jnp.float32(1.0) - u2))
    x = jnp.clip(sel_mean + noise, -1.0, 1.0)
    y = jnp.round((x + jnp.float32(1.0)) * jnp.float32(127.5))
    out_ref[...] = y.astype(jnp.int32)


def kernel(l):
    B, C, T = l.shape
    n = C // 3
    BW, TW = 8, 2048
    lt = jnp.transpose(l, (1, 0, 2))
    out = pl.pallas_call(
        functools.partial(_body, n=n, T=T, BW=BW, TW=TW),
        grid=(B // BW, T // TW),
        in_specs=[pl.BlockSpec((C, BW, TW), lambda i, j: (0, i, j))],
        out_specs=pl.BlockSpec((BW, TW), lambda i, j: (i, j)),
        out_shape=jax.ShapeDtypeStruct((B, T), jnp.int32),
    )(lt)
    return out


# speedup vs baseline: 2.0843x; 1.0386x over previous
import functools
import numpy as np
import jax
import jax.numpy as jnp
from jax import lax
from jax.experimental import pallas as pl


def _np_threefry2x32(k0, k1, x0, x1):
    def rotl(x, d):
        return ((x << np.uint32(d)) | (x >> np.uint32(32 - d))).astype(np.uint32)
    ks = [np.uint32(k0), np.uint32(k1),
          np.uint32(np.uint32(k0) ^ np.uint32(k1) ^ np.uint32(0x1BD11BDA))]
    x0 = (x0 + ks[0]).astype(np.uint32)
    x1 = (x1 + ks[1]).astype(np.uint32)
    rots = [[13, 15, 26, 6], [17, 29, 16, 24]]
    for i in range(5):
        for r in rots[i % 2]:
            x0 = (x0 + x1).astype(np.uint32)
            x1 = rotl(x1, r)
            x1 = (x0 ^ x1).astype(np.uint32)
        x0 = (x0 + ks[(i + 1) % 3]).astype(np.uint32)
        x1 = (x1 + ks[(i + 2) % 3] + np.uint32(i + 1)).astype(np.uint32)
    return x0, x1


def _child_key(seed_hi, seed_lo, i):
    a, b = _np_threefry2x32(seed_hi, seed_lo,
                            np.array([0], np.uint32), np.array([i], np.uint32))
    return int(a[0]), int(b[0])


_K1 = _child_key(0, 42, 0)
_K2 = _child_key(0, 42, 1)

_TINY = float(np.finfo(np.float32).tiny)
_U2_MIN = np.float32(1e-5)
_U2_SPAN = np.float32(np.float32(1.0 - 1e-5) - np.float32(1e-5))

_ROT = ((13, 15, 26, 6), (17, 29, 16, 24))


def _wrap_i32(v):
    v &= 0xFFFFFFFF
    return v - (1 << 32) if v >= (1 << 31) else v


def _rotl(x, r):
    return lax.shift_left(x, jnp.int32(r)) | lax.shift_right_logical(x, jnp.int32(32 - r))


def _key_sched(kpair):
    k0, k1 = kpair
    ks = (_wrap_i32(k0), _wrap_i32(k1), _wrap_i32(k0 ^ k1 ^ 0x1BD11BDA))
    inj = [(ks[(i + 1) % 3], _wrap_i32(ks[(i + 2) % 3] + i + 1)) for i in range(5)]
    return ks[0], ks[1], inj


_SCHED1 = _key_sched(_K1)
_SCHED2 = _key_sched(_K2)


def _threefry_bits(sched, x1):
    ks0, _ks1, inj = sched
    x0 = jnp.full_like(x1, jnp.int32(ks0))
    for i in range(5):
        for r in _ROT[i % 2]:
            x0 = x0 + x1
            x1 = _rotl(x1, r)
            x1 = x0 ^ x1
        x0 = x0 + jnp.int32(inj[i][0])
        x1 = x1 + jnp.int32(inj[i][1])
    return x0 ^ x1


def _bits_to_unit(bits):
    fb = lax.shift_right_logical(bits, jnp.int32(9)) | jnp.int32(0x3F800000)
    return lax.bitcast_convert_type(fb, jnp.float32) - jnp.float32(1.0)


def _body(l_ref, cnt1_ref, cnt2_ref, out_ref, *, n, T, BW, TW):
    i = pl.program_id(0)
    j = pl.program_id(1)
    lb = l_ref[...]

    base1 = (i * BW * n) * T + j * TW
    x1 = cnt1_ref[...] + base1
    bits = _threefry_bits(_SCHED1, x1)
    ug = jnp.maximum(_bits_to_unit(bits), jnp.float32(_TINY))
    gum = -jnp.log(-jnp.log(ug))

    bv = lb[0] + gum[0]
    bm = lb[n]
    bl = lb[2 * n]
    for g in range(1, n):
        vg = lb[g] + gum[g]
        upd = vg > bv
        bv = jnp.where(upd, vg, bv)
        bm = jnp.where(upd, lb[n + g], bm)
        bl = jnp.where(upd, lb[2 * n + g], bl)

    sel_mean = jnp.tanh(bm)
    sel_lv = jnp.float32(-7.0) * jax.nn.sigmoid(bl)

    base2 = (i * BW) * T + j * TW
    bits2 = _threefry_bits(_SCHED2, cnt2_ref[...] + base2)
    u2 = jnp.maximum(_U2_MIN, _bits_to_unit(bits2) * _U2_SPAN + _U2_MIN)

    noise = jnp.exp(sel_lv) * (jnp.log(u2) - jnp.log(jnp.float32(1.0) - u2))
    x = jnp.clip(sel_mean + noise, -1.0, 1.0)
    y = jnp.round((x + jnp.float32(1.0)) * jnp.float32(127.5))
    out_ref[...] = y.astype(jnp.int32)


def _counter_consts(n, T, BW, TW):
    g = np.arange(n, dtype=np.int64)[:, None, None]
    b = np.arange(BW, dtype=np.int64)[None, :, None]
    c = np.arange(TW, dtype=np.int64)[None, None, :]
    cnt1 = (b * n + g) * T + c + _SCHED1[1]
    cnt2 = (np.arange(BW, dtype=np.int64)[:, None] * T
            + np.arange(TW, dtype=np.int64)[None, :] + _SCHED2[1])
    wrap = lambda a: ((a & 0xFFFFFFFF) ^ (1 << 31)) - (1 << 31)
    return (jnp.asarray(wrap(cnt1), jnp.int32), jnp.asarray(wrap(cnt2), jnp.int32))


def kernel(l):
    B, C, T = l.shape
    n = C // 3
    BW, TW = 8, 2048
    lt = jnp.transpose(l, (1, 0, 2))
    cnt1, cnt2 = _counter_consts(n, T, BW, TW)
    out = pl.pallas_call(
        functools.partial(_body, n=n, T=T, BW=BW, TW=TW),
        grid=(B // BW, T // TW),
        in_specs=[
            pl.BlockSpec((C, BW, TW), lambda i, j: (0, i, j)),
            pl.BlockSpec((n, BW, TW), lambda i, j: (0, 0, 0)),
            pl.BlockSpec((BW, TW), lambda i, j: (0, 0)),
        ],
        out_specs=pl.BlockSpec((BW, TW), lambda i, j: (i, j)),
        out_shape=jax.ShapeDtypeStruct((B, T), jnp.int32),
    )(lt, cnt1, cnt2)
    return out


# speedup vs baseline: 2.0853x; 1.0004x over previous
import functools
import numpy as np
import jax
import jax.numpy as jnp
from jax import lax
from jax.experimental import pallas as pl


def _np_threefry2x32(k0, k1, x0, x1):
    def rotl(x, d):
        return ((x << np.uint32(d)) | (x >> np.uint32(32 - d))).astype(np.uint32)
    ks = [np.uint32(k0), np.uint32(k1),
          np.uint32(np.uint32(k0) ^ np.uint32(k1) ^ np.uint32(0x1BD11BDA))]
    x0 = (x0 + ks[0]).astype(np.uint32)
    x1 = (x1 + ks[1]).astype(np.uint32)
    rots = [[13, 15, 26, 6], [17, 29, 16, 24]]
    for i in range(5):
        for r in rots[i % 2]:
            x0 = (x0 + x1).astype(np.uint32)
            x1 = rotl(x1, r)
            x1 = (x0 ^ x1).astype(np.uint32)
        x0 = (x0 + ks[(i + 1) % 3]).astype(np.uint32)
        x1 = (x1 + ks[(i + 2) % 3] + np.uint32(i + 1)).astype(np.uint32)
    return x0, x1


def _child_key(seed_hi, seed_lo, i):
    a, b = _np_threefry2x32(seed_hi, seed_lo,
                            np.array([0], np.uint32), np.array([i], np.uint32))
    return int(a[0]), int(b[0])


_K1 = _child_key(0, 42, 0)
_K2 = _child_key(0, 42, 1)

_TINY = float(np.finfo(np.float32).tiny)
_U2_MIN = np.float32(1e-5)
_U2_SPAN = np.float32(np.float32(1.0 - 1e-5) - np.float32(1e-5))

_ROT = ((13, 15, 26, 6), (17, 29, 16, 24))


def _wrap_i32(v):
    v &= 0xFFFFFFFF
    return v - (1 << 32) if v >= (1 << 31) else v


def _rotl(x, r):
    return lax.shift_left(x, jnp.int32(r)) | lax.shift_right_logical(x, jnp.int32(32 - r))


def _key_sched(kpair):
    k0, k1 = kpair
    ks = (_wrap_i32(k0), _wrap_i32(k1), _wrap_i32(k0 ^ k1 ^ 0x1BD11BDA))
    inj = [(ks[(i + 1) % 3], _wrap_i32(ks[(i + 2) % 3] + i + 1)) for i in range(5)]
    return ks[0], ks[1], inj


_SCHED1 = _key_sched(_K1)
_SCHED2 = _key_sched(_K2)


def _threefry_bits(sched, x1):
    ks0, _ks1, inj = sched
    x0 = jnp.full_like(x1, jnp.int32(ks0))
    for i in range(5):
        for r in _ROT[i % 2]:
            x0 = x0 + x1
            x1 = _rotl(x1, r)
            x1 = x0 ^ x1
        x0 = x0 + jnp.int32(inj[i][0])
        x1 = x1 + jnp.int32(inj[i][1])
    return x0 ^ x1


def _bits_to_unit(bits):
    fb = lax.shift_right_logical(bits, jnp.int32(9)) | jnp.int32(0x3F800000)
    return lax.bitcast_convert_type(fb, jnp.float32) - jnp.float32(1.0)


def _body(l_ref, cnt1_ref, cnt2_ref, out_ref, *, n, T, BW, TW):
    i = pl.program_id(0)
    j = pl.program_id(1)
    lb = l_ref[...]

    base1 = (i * BW * n) * T + j * TW
    x1 = cnt1_ref[...] + base1
    bits = _threefry_bits(_SCHED1, x1)
    ug = jnp.maximum(_bits_to_unit(bits), jnp.float32(_TINY))
    gum = -jnp.log(-jnp.log(ug))

    bv = lb[0] + gum[0]
    bm = lb[n]
    bl = lb[2 * n]
    for g in range(1, n):
        vg = lb[g] + gum[g]
        upd = vg > bv
        bv = jnp.where(upd, vg, bv)
        bm = jnp.where(upd, lb[n + g], bm)
        bl = jnp.where(upd, lb[2 * n + g], bl)

    sel_mean = jnp.tanh(bm)
    sel_lv = jnp.float32(-7.0) * jax.nn.sigmoid(bl)

    base2 = (i * BW) * T + j * TW
    bits2 = _threefry_bits(_SCHED2, cnt2_ref[...] + base2)
    u2 = jnp.maximum(_U2_MIN, _bits_to_unit(bits2) * _U2_SPAN + _U2_MIN)

    noise = jnp.exp(sel_lv) * (jnp.log(u2) - jnp.log(jnp.float32(1.0) - u2))
    x = jnp.clip(sel_mean + noise, -1.0, 1.0)
    y = jnp.round((x + jnp.float32(1.0)) * jnp.float32(127.5))
    out_ref[...] = y.astype(jnp.int32)


def _counter_consts(n, T, BW, TW):
    g = np.arange(n, dtype=np.int64)[:, None, None]
    b = np.arange(BW, dtype=np.int64)[None, :, None]
    c = np.arange(TW, dtype=np.int64)[None, None, :]
    cnt1 = (b * n + g) * T + c + _SCHED1[1]
    cnt2 = (np.arange(BW, dtype=np.int64)[:, None] * T
            + np.arange(TW, dtype=np.int64)[None, :] + _SCHED2[1])
    wrap = lambda a: ((a & 0xFFFFFFFF) ^ (1 << 31)) - (1 << 31)
    return (jnp.asarray(wrap(cnt1), jnp.int32), jnp.asarray(wrap(cnt2), jnp.int32))


def kernel(l):
    B, C, T = l.shape
    n = C // 3
    BW, TW = 8, 1024
    lt = jnp.transpose(l, (1, 0, 2))
    cnt1, cnt2 = _counter_consts(n, T, BW, TW)
    out = pl.pallas_call(
        functools.partial(_body, n=n, T=T, BW=BW, TW=TW),
        grid=(B // BW, T // TW),
        in_specs=[
            pl.BlockSpec((C, BW, TW), lambda i, j: (0, i, j)),
            pl.BlockSpec((n, BW, TW), lambda i, j: (0, 0, 0)),
            pl.BlockSpec((BW, TW), lambda i, j: (0, 0)),
        ],
        out_specs=pl.BlockSpec((BW, TW), lambda i, j: (i, j)),
        out_shape=jax.ShapeDtypeStruct((B, T), jnp.int32),
    )(lt, cnt1, cnt2)
    return out
